# Initial kernel scaffold; baseline (speedup 1.0000x reference)
import jax
import jax.numpy as jnp
from jax.experimental import pallas as pl


def kernel(x, edge_index, edge_attr, params):
    raise NotImplementedError("write your pallas kernel here")


# speedup vs baseline: 1.7623x; 1.7623x over previous
import functools

import jax
import jax.numpy as jnp
from jax import lax
from jax.experimental import pallas as pl
from jax.experimental.pallas import tpu as pltpu
from jax.experimental.pallas import tpu_sc as plsc

N = 10000
E = 320000
D = 128
NB = 1000
EB = 2000

NW = 32
EW = E // NW
C = 80
NCH = EW // C
NPAD = 10240
SR = NPAD // 16
ZR = 128

_PREC = jax.lax.Precision.HIGHEST


def _dot(a, b):
    return jnp.dot(a, b, preferred_element_type=jnp.float32, precision=_PREC)


def _ln(h, g, b):
    mu = jnp.mean(h, axis=-1, keepdims=True)
    d = h - mu
    var = jnp.mean(d * d, axis=-1, keepdims=True)
    return d * lax.rsqrt(var + 1e-5) * g + b


def _row(v):
    return v.reshape(1, -1)


def _full(shape):
    return pl.BlockSpec(shape, lambda i: (0,) * len(shape))


def _rows(blk, width):
    return pl.BlockSpec((blk, width), lambda i: (i, 0))


def _mlp2_ln(x, w1, b1, w2, b2, g, b, blk):
    n, din = x.shape
    dout = w2.shape[1]

    def body(x_r, w1_r, b1_r, w2_r, b2_r, g_r, b_r, o_r):
        h = jnp.maximum(_dot(x_r[...], w1_r[...]) + b1_r[...], 0.0)
        h = _dot(h, w2_r[...]) + b2_r[...]
        o_r[...] = _ln(h, g_r[...], b_r[...])

    return pl.pallas_call(
        body,
        grid=(n // blk,),
        in_specs=[_rows(blk, din), _full(w1.shape), _full((1, D)),
                  _full(w2.shape), _full((1, D)), _full((1, D)), _full((1, D))],
        out_specs=_rows(blk, dout),
        out_shape=jax.ShapeDtypeStruct((n, dout), jnp.float32),
    )(x, w1, _row(b1), w2, _row(b2), _row(g), _row(b))


def _pre_tables(h, w1a, w1b, b1):
    def body(h_r, wa_r, wb_r, b1_r, oa_r, ob_r):
        hv = h_r[...]
        oa_r[...] = _dot(hv, wa_r[...])
        ob_r[...] = _dot(hv, wb_r[...]) + b1_r[...]

    return pl.pallas_call(
        body,
        grid=(N // NB,),
        in_specs=[_rows(NB, D), _full((D, D)), _full((D, D)), _full((1, D))],
        out_specs=[_rows(NB, D), _rows(NB, D)],
        out_shape=[jax.ShapeDtypeStruct((N, D), jnp.float32)] * 2,
    )(h, w1a, w1b, _row(b1))


def _edge_mlp(gsum, e, w1c, w2, b2, g, b):
    def body(g_r, e_r, w1c_r, w2_r, b2_r, gg_r, bb_r, o_r):
        x = jnp.maximum(g_r[...] + _dot(e_r[...], w1c_r[...]), 0.0)
        m = jnp.maximum(_dot(x, w2_r[...]) + b2_r[...], 0.0)
        o_r[...] = _ln(m, gg_r[...], bb_r[...])

    return pl.pallas_call(
        body,
        grid=(E // EB,),
        in_specs=[_rows(EB, D), _rows(EB, D), _full((D, D)),
                  _full((D, D)), _full((1, D)), _full((1, D)), _full((1, D))],
        out_specs=_rows(EB, D),
        out_shape=jax.ShapeDtypeStruct((E, D), jnp.float32),
    )(gsum, e, w1c, w2, _row(b2), _row(g), _row(b))


def _node_update(h, p0, p1, w3a, w3b, b3, w4, b4, g, b):
    def body(h_r, p0_r, p1_r, wa_r, wb_r, b3_r, w4_r, b4_r, gg_r, bb_r, o_r):
        hv = h_r[...]
        agg = p0_r[...] + p1_r[...]
        nu = jnp.maximum(_dot(hv, wa_r[...]) + _dot(agg, wb_r[...]) + b3_r[...], 0.0)
        nu = _dot(nu, w4_r[...]) + b4_r[...]
        o_r[...] = _ln(hv + nu, gg_r[...], bb_r[...])

    return pl.pallas_call(
        body,
        grid=(N // NB,),
        in_specs=[_rows(NB, D), _rows(NB, D), _rows(NB, D), _full((D, D)),
                  _full((D, D)), _full((1, D)), _full((D, D)), _full((1, D)),
                  _full((1, D)), _full((1, D))],
        out_specs=_rows(NB, D),
        out_shape=jax.ShapeDtypeStruct((N, D), jnp.float32),
    )(h, p0, p1, w3a, w3b, _row(b3), w4, _row(b4), _row(g), _row(b))


def _decoder(h, w1, b1, w2, b2):
    def body(h_r, w1_r, b1_r, w2_r, b2_r, o_r):
        x = jnp.maximum(_dot(h_r[...], w1_r[...]) + b1_r[...], 0.0)
        o_r[...] = _dot(x, w2_r[...]) + b2_r[...]

    return pl.pallas_call(
        body,
        grid=(N // NB,),
        in_specs=[_rows(NB, D), _full((D, D)), _full((1, D)),
                  _full((D, 3)), _full((1, 3))],
        out_specs=_rows(NB, 3),
        out_shape=jax.ShapeDtypeStruct((N, 3), jnp.float32),
    )(h, w1, _row(b1), w2, _row(b2))


_SC_MESH = plsc.VectorSubcoreMesh(core_axis_name="c", subcore_axis_name="s")


def _gather_add(hA, hB, sidx, ridx):
    @functools.partial(
        pl.kernel,
        out_type=jax.ShapeDtypeStruct((E, D), jnp.float32),
        mesh=_SC_MESH,
        scratch_types=[
            pltpu.VMEM((C,), jnp.int32),
            pltpu.VMEM((C,), jnp.int32),
            pltpu.VMEM((C, D), jnp.float32),
            pltpu.VMEM((C, D), jnp.float32),
            pltpu.SemaphoreType.DMA,
            pltpu.SemaphoreType.DMA,
        ],
    )
    def k(hA_h, hB_h, s_h, r_h, out_h, sv, rv, ra, rb, sa, sb):
        wid = lax.axis_index("s") * 2 + lax.axis_index("c")
        base0 = wid * EW

        def chunk(i, carry):
            base = base0 + i * C
            pltpu.sync_copy(s_h.at[pl.ds(base, C)], sv)
            pltpu.sync_copy(r_h.at[pl.ds(base, C)], rv)
            ca = pltpu.async_copy(hA_h.at[sv], ra, sa)
            cb = pltpu.async_copy(hB_h.at[rv], rb, sb)
            ca.wait()
            cb.wait()

            def addg(j, _):
                r = j // 8
                cc = (j % 8) * 16
                ra[r, pl.ds(cc, 16)] = ra[r, pl.ds(cc, 16)] + rb[r, pl.ds(cc, 16)]
                return _

            lax.fori_loop(0, C * 8, addg, 0)
            pltpu.sync_copy(ra, out_h.at[pl.ds(base, C)])
            return carry

        lax.fori_loop(0, NCH, chunk, 0)

    return k(hA, hB, sidx, ridx)


def _scatter_add(m, ridx):
    @functools.partial(
        pl.kernel,
        out_type=[jax.ShapeDtypeStruct((NPAD, D), jnp.float32)] * 2,
        mesh=_SC_MESH,
        scratch_types=[
            pltpu.VMEM((C,), jnp.int32),
            pltpu.VMEM((C, D), jnp.float32),
            pltpu.VMEM((ZR, D), jnp.float32),
            pltpu.VMEM_SHARED((NPAD, D), jnp.float32),
            pltpu.SemaphoreType.DMA,
        ],
    )
    def k(m_h, r_h, o0, o1, iv, mv, zv, acc, sem):
        c = lax.axis_index("c")
        s = lax.axis_index("s")
        wid = s * 2 + c

        def zst(j, _):
            zv[j // 8, pl.ds((j % 8) * 16, 16)] = jnp.zeros((16,), jnp.float32)
            return _

        lax.fori_loop(0, ZR * 8, zst, 0)
        for t in range(SR // ZR):
            pltpu.sync_copy(zv, acc.at[pl.ds(s * SR + t * ZR, ZR)])
        plsc.subcore_barrier()

        def chunk(i, carry):
            base = wid * EW + i * C
            pltpu.sync_copy(r_h.at[pl.ds(base, C)], iv)
            pltpu.sync_copy(m_h.at[pl.ds(base, C)], mv)
            pltpu.sync_copy(mv, acc.at[iv], add=True)
            return carry

        lax.fori_loop(0, NCH, chunk, 0)
        plsc.subcore_barrier()

        @pl.when(c == 0)
        def _():
            pltpu.sync_copy(acc.at[pl.ds(s * SR, SR)], o0.at[pl.ds(s * SR, SR)])

        @pl.when(c == 1)
        def _():
            pltpu.sync_copy(acc.at[pl.ds(s * SR, SR)], o1.at[pl.ds(s * SR, SR)])

    return k(m, ridx)


def kernel(x, edge_index, edge_attr, params):
    senders = edge_index[0].astype(jnp.int32)
    receivers = edge_index[1].astype(jnp.int32)

    ne0, ne1 = params["node_enc"]
    h = _mlp2_ln(x, ne0["W"], ne0["b"], ne1["W"], ne1["b"],
                 params["node_enc_ln"]["g"], params["node_enc_ln"]["b"], NB)
    ee0, ee1 = params["edge_enc"]
    e = _mlp2_ln(edge_attr, ee0["W"], ee0["b"], ee1["W"], ee1["b"],
                 params["edge_enc_ln"]["g"], params["edge_enc_ln"]["b"], EB)

    for lp in params["layers"]:
        w1 = lp["edge_mlp"][0]["W"]
        b1 = lp["edge_mlp"][0]["b"]
        w1a, w1b, w1c = w1[:D], w1[D:2 * D], w1[2 * D:]
        hA, hB = _pre_tables(h, w1a, w1b, b1)
        gsum = _gather_add(hA, hB, senders, receivers)
        m = _edge_mlp(gsum, e, w1c, lp["edge_mlp"][1]["W"], lp["edge_mlp"][1]["b"],
                      lp["edge_norm"]["g"], lp["edge_norm"]["b"])
        p0, p1 = _scatter_add(m, receivers)
        w3 = lp["node_mlp"][0]["W"]
        h = _node_update(h, p0, p1, w3[:D], w3[D:], lp["node_mlp"][0]["b"],
                         lp["node_mlp"][1]["W"], lp["node_mlp"][1]["b"],
                         lp["node_norm"]["g"], lp["node_norm"]["b"])

    d0, d1 = params["dec"]
    return _decoder(h, d0["W"], d0["b"], d1["W"], d1["b"])


# speedup vs baseline: 2.3209x; 1.3170x over previous
import functools

import jax
import jax.numpy as jnp
from jax import lax
from jax.experimental import pallas as pl
from jax.experimental.pallas import tpu as pltpu
from jax.experimental.pallas import tpu_sc as plsc

N = 10000
E = 320000
D = 128
NB = 1000
EB = 2000

NW = 32
EW = E // NW
C = 80
NCH = EW // C
NPAD = 10240
SR = NPAD // 16
ZR = 128

_PREC = jax.lax.Precision.HIGHEST


def _dot(a, b):
    return jnp.dot(a, b, preferred_element_type=jnp.float32, precision=_PREC)


def _ln(h, g, b):
    mu = jnp.mean(h, axis=-1, keepdims=True)
    d = h - mu
    var = jnp.mean(d * d, axis=-1, keepdims=True)
    return d * lax.rsqrt(var + 1e-5) * g + b


def _row(v):
    return v.reshape(1, -1)


def _full(shape):
    return pl.BlockSpec(shape, lambda i: (0,) * len(shape))


def _rows(blk, width):
    return pl.BlockSpec((blk, width), lambda i: (i, 0))


def _mlp2_ln(x, w1, b1, w2, b2, g, b, blk):
    n, din = x.shape
    dout = w2.shape[1]

    def body(x_r, w1_r, b1_r, w2_r, b2_r, g_r, b_r, o_r):
        h = jnp.maximum(_dot(x_r[...], w1_r[...]) + b1_r[...], 0.0)
        h = _dot(h, w2_r[...]) + b2_r[...]
        o_r[...] = _ln(h, g_r[...], b_r[...])

    return pl.pallas_call(
        body,
        grid=(n // blk,),
        in_specs=[_rows(blk, din), _full(w1.shape), _full((1, D)),
                  _full(w2.shape), _full((1, D)), _full((1, D)), _full((1, D))],
        out_specs=_rows(blk, dout),
        out_shape=jax.ShapeDtypeStruct((n, dout), jnp.float32),
    )(x, w1, _row(b1), w2, _row(b2), _row(g), _row(b))


def _pre_tables(h, w1a, w1b, b1):
    def body(h_r, wa_r, wb_r, b1_r, oa_r, ob_r):
        hv = h_r[...]
        oa_r[...] = _dot(hv, wa_r[...])
        ob_r[...] = _dot(hv, wb_r[...]) + b1_r[...]

    return pl.pallas_call(
        body,
        grid=(N // NB,),
        in_specs=[_rows(NB, D), _full((D, D)), _full((D, D)), _full((1, D))],
        out_specs=[_rows(NB, D), _rows(NB, D)],
        out_shape=[jax.ShapeDtypeStruct((N, D), jnp.float32)] * 2,
    )(h, w1a, w1b, _row(b1))


def _edge_mlp(gsum, e, w1c, w2, b2, g, b):
    def body(g_r, e_r, w1c_r, w2_r, b2_r, gg_r, bb_r, o_r):
        x = jnp.maximum(g_r[...] + _dot(e_r[...], w1c_r[...]), 0.0)
        m = jnp.maximum(_dot(x, w2_r[...]) + b2_r[...], 0.0)
        o_r[...] = _ln(m, gg_r[...], bb_r[...])

    return pl.pallas_call(
        body,
        grid=(E // EB,),
        in_specs=[_rows(EB, D), _rows(EB, D), _full((D, D)),
                  _full((D, D)), _full((1, D)), _full((1, D)), _full((1, D))],
        out_specs=_rows(EB, D),
        out_shape=jax.ShapeDtypeStruct((E, D), jnp.float32),
    )(gsum, e, w1c, w2, _row(b2), _row(g), _row(b))


def _node_update(h, p0, p1, w3a, w3b, b3, w4, b4, g, b):
    def body(h_r, p0_r, p1_r, wa_r, wb_r, b3_r, w4_r, b4_r, gg_r, bb_r, o_r):
        hv = h_r[...]
        agg = p0_r[...] + p1_r[...]
        nu = jnp.maximum(_dot(hv, wa_r[...]) + _dot(agg, wb_r[...]) + b3_r[...], 0.0)
        nu = _dot(nu, w4_r[...]) + b4_r[...]
        o_r[...] = _ln(hv + nu, gg_r[...], bb_r[...])

    return pl.pallas_call(
        body,
        grid=(N // NB,),
        in_specs=[_rows(NB, D), _rows(NB, D), _rows(NB, D), _full((D, D)),
                  _full((D, D)), _full((1, D)), _full((D, D)), _full((1, D)),
                  _full((1, D)), _full((1, D))],
        out_specs=_rows(NB, D),
        out_shape=jax.ShapeDtypeStruct((N, D), jnp.float32),
    )(h, p0, p1, w3a, w3b, _row(b3), w4, _row(b4), _row(g), _row(b))


def _decoder(h, w1, b1, w2, b2):
    def body(h_r, w1_r, b1_r, w2_r, b2_r, o_r):
        x = jnp.maximum(_dot(h_r[...], w1_r[...]) + b1_r[...], 0.0)
        o_r[...] = _dot(x, w2_r[...]) + b2_r[...]

    return pl.pallas_call(
        body,
        grid=(N // NB,),
        in_specs=[_rows(NB, D), _full((D, D)), _full((1, D)),
                  _full((D, 3)), _full((1, 3))],
        out_specs=_rows(NB, 3),
        out_shape=jax.ShapeDtypeStruct((N, 3), jnp.float32),
    )(h, w1, _row(b1), w2, _row(b2))


_SC_MESH = plsc.VectorSubcoreMesh(core_axis_name="c", subcore_axis_name="s")


def _gather_add(hA, hB, sidx, ridx):
    NPAIR = (NCH + 1) // 2

    @functools.partial(
        pl.kernel,
        out_type=jax.ShapeDtypeStruct((E, D), jnp.float32),
        mesh=_SC_MESH,
        scratch_types=[
            pltpu.VMEM((EW,), jnp.int32),
            pltpu.VMEM((EW,), jnp.int32),
            pltpu.VMEM((C, D), jnp.float32),
            pltpu.VMEM((C, D), jnp.float32),
            pltpu.VMEM((C, D), jnp.float32),
            pltpu.VMEM((C, D), jnp.float32),
            pltpu.SemaphoreType.DMA,
            pltpu.SemaphoreType.DMA,
            pltpu.SemaphoreType.DMA,
            pltpu.SemaphoreType.DMA,
        ],
    )
    def k(hA_h, hB_h, s_h, r_h, out_h,
          sv, rv, ra0, rb0, ra1, rb1, sa0, sb0, sa1, sb1):
        wid = lax.axis_index("s") * 2 + lax.axis_index("c")
        base0 = wid * EW
        pltpu.sync_copy(s_h.at[pl.ds(base0, EW)], sv)
        pltpu.sync_copy(r_h.at[pl.ds(base0, EW)], rv)

        def start(i, ra, rb, sa, sb):
            off = i * C
            pltpu.async_copy(hA_h.at[sv.at[pl.ds(off, C)]], ra, sa)
            pltpu.async_copy(hB_h.at[rv.at[pl.ds(off, C)]], rb, sb)

        def finish(i, ra, rb, sa, sb):
            pltpu.make_async_copy(hA_h.at[pl.ds(0, C)], ra, sa).wait()
            pltpu.make_async_copy(hB_h.at[pl.ds(0, C)], rb, sb).wait()

            def rowadd(r, carry):
                for q in range(8):
                    sl = pl.ds(q * 16, 16)
                    ra[r, sl] = ra[r, sl] + rb[r, sl]
                return carry

            lax.fori_loop(0, C, rowadd, 0)
            pltpu.sync_copy(ra, out_h.at[pl.ds(base0 + i * C, C)])

        start(0, ra0, rb0, sa0, sb0)
        start(1, ra1, rb1, sa1, sb1)

        def pair(j, carry):
            finish(2 * j, ra0, rb0, sa0, sb0)

            @pl.when(2 * j + 2 < NCH)
            def _():
                start(2 * j + 2, ra0, rb0, sa0, sb0)

            @pl.when(2 * j + 1 < NCH)
            def _():
                finish(2 * j + 1, ra1, rb1, sa1, sb1)

                @pl.when(2 * j + 3 < NCH)
                def _():
                    start(2 * j + 3, ra1, rb1, sa1, sb1)

            return carry

        lax.fori_loop(0, NPAIR, pair, 0)

    return k(hA, hB, sidx, ridx)


def _scatter_add(m, ridx):
    NPAIR = (NCH + 1) // 2

    @functools.partial(
        pl.kernel,
        out_type=[jax.ShapeDtypeStruct((NPAD, D), jnp.float32)] * 2,
        mesh=_SC_MESH,
        scratch_types=[
            pltpu.VMEM((C,), jnp.int32),
            pltpu.VMEM((C,), jnp.int32),
            pltpu.VMEM((C, D), jnp.float32),
            pltpu.VMEM((C, D), jnp.float32),
            pltpu.VMEM((ZR, D), jnp.float32),
            pltpu.VMEM_SHARED((NPAD, D), jnp.float32),
            pltpu.SemaphoreType.DMA,
            pltpu.SemaphoreType.DMA,
            pltpu.SemaphoreType.DMA,
            pltpu.SemaphoreType.DMA,
        ],
    )
    def k(m_h, r_h, o0, o1, iv0, iv1, mv0, mv1, zv, acc, si0, si1, sm0, sm1):
        c = lax.axis_index("c")
        s = lax.axis_index("s")
        wid = s * 2 + c
        base0 = wid * EW

        def start(i, iv, mv, si, sm):
            base = base0 + i * C
            pltpu.async_copy(r_h.at[pl.ds(base, C)], iv, si)
            pltpu.async_copy(m_h.at[pl.ds(base, C)], mv, sm)

        start(0, iv0, mv0, si0, sm0)
        start(1, iv1, mv1, si1, sm1)

        def zst(j, _):
            zv[j // 8, pl.ds((j % 8) * 16, 16)] = jnp.zeros((16,), jnp.float32)
            return _

        lax.fori_loop(0, ZR * 8, zst, 0)
        for t in range(SR // ZR):
            pltpu.sync_copy(zv, acc.at[pl.ds(s * SR + t * ZR, ZR)])
        plsc.subcore_barrier()

        def finish(i, iv, mv, si, sm):
            pltpu.make_async_copy(r_h.at[pl.ds(0, C)], iv, si).wait()
            pltpu.make_async_copy(m_h.at[pl.ds(0, C)], mv, sm).wait()
            pltpu.sync_copy(mv, acc.at[iv], add=True)

        def pair(j, carry):
            finish(2 * j, iv0, mv0, si0, sm0)

            @pl.when(2 * j + 2 < NCH)
            def _():
                start(2 * j + 2, iv0, mv0, si0, sm0)

            @pl.when(2 * j + 1 < NCH)
            def _():
                finish(2 * j + 1, iv1, mv1, si1, sm1)

                @pl.when(2 * j + 3 < NCH)
                def _():
                    start(2 * j + 3, iv1, mv1, si1, sm1)

            return carry

        lax.fori_loop(0, NPAIR, pair, 0)
        plsc.subcore_barrier()

        @pl.when(c == 0)
        def _():
            pltpu.sync_copy(acc.at[pl.ds(s * SR, SR)], o0.at[pl.ds(s * SR, SR)])

        @pl.when(c == 1)
        def _():
            pltpu.sync_copy(acc.at[pl.ds(s * SR, SR)], o1.at[pl.ds(s * SR, SR)])

    return k(m, ridx)


def kernel(x, edge_index, edge_attr, params):
    senders = edge_index[0].astype(jnp.int32)
    receivers = edge_index[1].astype(jnp.int32)

    ne0, ne1 = params["node_enc"]
    h = _mlp2_ln(x, ne0["W"], ne0["b"], ne1["W"], ne1["b"],
                 params["node_enc_ln"]["g"], params["node_enc_ln"]["b"], NB)
    ee0, ee1 = params["edge_enc"]
    e = _mlp2_ln(edge_attr, ee0["W"], ee0["b"], ee1["W"], ee1["b"],
                 params["edge_enc_ln"]["g"], params["edge_enc_ln"]["b"], EB)

    for lp in params["layers"]:
        w1 = lp["edge_mlp"][0]["W"]
        b1 = lp["edge_mlp"][0]["b"]
        w1a, w1b, w1c = w1[:D], w1[D:2 * D], w1[2 * D:]
        hA, hB = _pre_tables(h, w1a, w1b, b1)
        gsum = _gather_add(hA, hB, senders, receivers)
        m = _edge_mlp(gsum, e, w1c, lp["edge_mlp"][1]["W"], lp["edge_mlp"][1]["b"],
                      lp["edge_norm"]["g"], lp["edge_norm"]["b"])
        p0, p1 = _scatter_add(m, receivers)
        w3 = lp["node_mlp"][0]["W"]
        h = _node_update(h, p0, p1, w3[:D], w3[D:], lp["node_mlp"][0]["b"],
                         lp["node_mlp"][1]["W"], lp["node_mlp"][1]["b"],
                         lp["node_norm"]["g"], lp["node_norm"]["b"])

    d0, d1 = params["dec"]
    return _decoder(h, d0["W"], d0["b"], d1["W"], d1["b"])


# speedup vs baseline: 2.9108x; 1.2542x over previous
import functools

import jax
import jax.numpy as jnp
from jax import lax
from jax.experimental import pallas as pl
from jax.experimental.pallas import tpu as pltpu
from jax.experimental.pallas import tpu_sc as plsc

N = 10000
E = 320000
D = 128
NB = 1000
EB = 4000

NW = 32
EW = E // NW
C = 80
NCH = EW // C
NPAD = 10240
SR = NPAD // 16
ZR = 128

_PREC = jax.lax.Precision.HIGHEST


def _dot(a, b):
    return jnp.dot(a, b, preferred_element_type=jnp.float32, precision=_PREC)


def _ln(h, g, b):
    mu = jnp.mean(h, axis=-1, keepdims=True)
    d = h - mu
    var = jnp.mean(d * d, axis=-1, keepdims=True)
    return d * lax.rsqrt(var + 1e-5) * g + b


def _row(v):
    return v.reshape(1, -1)


def _full(shape):
    return pl.BlockSpec(shape, lambda i: (0,) * len(shape))


def _rows(blk, width):
    return pl.BlockSpec((blk, width), lambda i: (i, 0))


def _node_encoder(x, w1, b1, w2, b2, g, b, w1a, w1b, bt):
    def body(x_r, w1_r, b1_r, w2_r, b2_r, g_r, b_r, wa_r, wb_r, bt_r,
             o_r, oa_r, ob_r):
        h = jnp.maximum(_dot(x_r[...], w1_r[...]) + b1_r[...], 0.0)
        h = _dot(h, w2_r[...]) + b2_r[...]
        h = _ln(h, g_r[...], b_r[...])
        o_r[...] = h
        oa_r[...] = _dot(h, wa_r[...])
        ob_r[...] = _dot(h, wb_r[...]) + bt_r[...]

    return pl.pallas_call(
        body,
        grid=(N // NB,),
        in_specs=[_rows(NB, D), _full((D, D)), _full((1, D)),
                  _full((D, D)), _full((1, D)), _full((1, D)), _full((1, D)),
                  _full((D, D)), _full((D, D)), _full((1, D))],
        out_specs=[_rows(NB, D)] * 3,
        out_shape=[jax.ShapeDtypeStruct((N, D), jnp.float32)] * 3,
    )(x, w1, _row(b1), w2, _row(b2), _row(g), _row(b), w1a, w1b, _row(bt))


def _edge_encoder(ea, w1, b1, w2, b2, g, b, wc1, wc2):
    def body(ea_r, w1_r, b1_r, w2_r, b2_r, g_r, b_r, c1_r, c2_r, o1_r, o2_r):
        t = jnp.maximum(_dot(ea_r[...], w1_r[...]) + b1_r[...], 0.0)
        e = _ln(_dot(t, w2_r[...]) + b2_r[...], g_r[...], b_r[...])
        o1_r[...] = _dot(e, c1_r[...])
        o2_r[...] = _dot(e, c2_r[...])

    return pl.pallas_call(
        body,
        grid=(E // EB,),
        in_specs=[_rows(EB, 16), _full((16, D)), _full((1, D)),
                  _full((D, D)), _full((1, D)), _full((1, D)), _full((1, D)),
                  _full((D, D)), _full((D, D))],
        out_specs=[_rows(EB, D)] * 2,
        out_shape=[jax.ShapeDtypeStruct((E, D), jnp.float32)] * 2,
    )(ea, w1, _row(b1), w2, _row(b2), _row(g), _row(b), wc1, wc2)


def _edge_mlp(gsum, ec, w2, b2, g, b):
    def body(g_r, e_r, w2_r, b2_r, gg_r, bb_r, o_r):
        x = jnp.maximum(g_r[...] + e_r[...], 0.0)
        m = jnp.maximum(_dot(x, w2_r[...]) + b2_r[...], 0.0)
        o_r[...] = _ln(m, gg_r[...], bb_r[...])

    return pl.pallas_call(
        body,
        grid=(E // EB,),
        in_specs=[_rows(EB, D), _rows(EB, D),
                  _full((D, D)), _full((1, D)), _full((1, D)), _full((1, D))],
        out_specs=_rows(EB, D),
        out_shape=jax.ShapeDtypeStruct((E, D), jnp.float32),
    )(gsum, ec, w2, _row(b2), _row(g), _row(b))


def _node_update(h, p0, p1, w3a, w3b, b3, w4, b4, g, b, w1a, w1b, bt):
    def body(h_r, p0_r, p1_r, wa_r, wb_r, b3_r, w4_r, b4_r, gg_r, bb_r,
             ta_r, tb_r, bt_r, o_r, oa_r, ob_r):
        hv = h_r[...]
        agg = p0_r[...] + p1_r[...]
        nu = jnp.maximum(_dot(hv, wa_r[...]) + _dot(agg, wb_r[...]) + b3_r[...], 0.0)
        nu = _dot(nu, w4_r[...]) + b4_r[...]
        hn = _ln(hv + nu, gg_r[...], bb_r[...])
        o_r[...] = hn
        oa_r[...] = _dot(hn, ta_r[...])
        ob_r[...] = _dot(hn, tb_r[...]) + bt_r[...]

    return pl.pallas_call(
        body,
        grid=(N // NB,),
        in_specs=[_rows(NB, D), _rows(NB, D), _rows(NB, D), _full((D, D)),
                  _full((D, D)), _full((1, D)), _full((D, D)), _full((1, D)),
                  _full((1, D)), _full((1, D)),
                  _full((D, D)), _full((D, D)), _full((1, D))],
        out_specs=[_rows(NB, D)] * 3,
        out_shape=[jax.ShapeDtypeStruct((N, D), jnp.float32)] * 3,
    )(h, p0, p1, w3a, w3b, _row(b3), w4, _row(b4), _row(g), _row(b),
      w1a, w1b, _row(bt))


def _node_update_dec(h, p0, p1, w3a, w3b, b3, w4, b4, g, b, wd1, bd1, wd2, bd2):
    def body(h_r, p0_r, p1_r, wa_r, wb_r, b3_r, w4_r, b4_r, gg_r, bb_r,
             d1_r, e1_r, d2_r, e2_r, o_r):
        hv = h_r[...]
        agg = p0_r[...] + p1_r[...]
        nu = jnp.maximum(_dot(hv, wa_r[...]) + _dot(agg, wb_r[...]) + b3_r[...], 0.0)
        nu = _dot(nu, w4_r[...]) + b4_r[...]
        hn = _ln(hv + nu, gg_r[...], bb_r[...])
        z = jnp.maximum(_dot(hn, d1_r[...]) + e1_r[...], 0.0)
        o_r[...] = _dot(z, d2_r[...]) + e2_r[...]

    return pl.pallas_call(
        body,
        grid=(N // NB,),
        in_specs=[_rows(NB, D), _rows(NB, D), _rows(NB, D), _full((D, D)),
                  _full((D, D)), _full((1, D)), _full((D, D)), _full((1, D)),
                  _full((1, D)), _full((1, D)),
                  _full((D, D)), _full((1, D)), _full((D, 3)), _full((1, 3))],
        out_specs=_rows(NB, 3),
        out_shape=jax.ShapeDtypeStruct((N, 3), jnp.float32),
    )(h, p0, p1, w3a, w3b, _row(b3), w4, _row(b4), _row(g), _row(b),
      wd1, _row(bd1), wd2, _row(bd2))


_SC_MESH = plsc.VectorSubcoreMesh(core_axis_name="c", subcore_axis_name="s")


def _gather_add(hA, hB, sidx, ridx):
    NPAIR = (NCH + 1) // 2

    @functools.partial(
        pl.kernel,
        out_type=jax.ShapeDtypeStruct((E, D), jnp.float32),
        mesh=_SC_MESH,
        scratch_types=[
            pltpu.VMEM((EW,), jnp.int32),
            pltpu.VMEM((EW,), jnp.int32),
            pltpu.VMEM((C, D), jnp.float32),
            pltpu.VMEM((C, D), jnp.float32),
            pltpu.VMEM((C, D), jnp.float32),
            pltpu.VMEM((C, D), jnp.float32),
            pltpu.SemaphoreType.DMA,
            pltpu.SemaphoreType.DMA,
            pltpu.SemaphoreType.DMA,
            pltpu.SemaphoreType.DMA,
        ],
    )
    def k(hA_h, hB_h, s_h, r_h, out_h,
          sv, rv, ra0, rb0, ra1, rb1, sa0, sb0, sa1, sb1):
        wid = lax.axis_index("s") * 2 + lax.axis_index("c")
        base0 = wid * EW
        pltpu.sync_copy(s_h.at[pl.ds(base0, EW)], sv)
        pltpu.sync_copy(r_h.at[pl.ds(base0, EW)], rv)

        def start(i, ra, rb, sa, sb):
            off = i * C
            pltpu.async_copy(hA_h.at[sv.at[pl.ds(off, C)]], ra, sa)
            pltpu.async_copy(hB_h.at[rv.at[pl.ds(off, C)]], rb, sb)

        def finish(i, ra, rb, sa, sb):
            pltpu.make_async_copy(hA_h.at[pl.ds(0, C)], ra, sa).wait()
            pltpu.make_async_copy(hB_h.at[pl.ds(0, C)], rb, sb).wait()

            def rowadd(r, carry):
                for q in range(8):
                    sl = pl.ds(q * 16, 16)
                    ra[r, sl] = ra[r, sl] + rb[r, sl]
                return carry

            lax.fori_loop(0, C, rowadd, 0)
            pltpu.sync_copy(ra, out_h.at[pl.ds(base0 + i * C, C)])

        start(0, ra0, rb0, sa0, sb0)
        start(1, ra1, rb1, sa1, sb1)

        def pair(j, carry):
            finish(2 * j, ra0, rb0, sa0, sb0)

            @pl.when(2 * j + 2 < NCH)
            def _():
                start(2 * j + 2, ra0, rb0, sa0, sb0)

            @pl.when(2 * j + 1 < NCH)
            def _():
                finish(2 * j + 1, ra1, rb1, sa1, sb1)

                @pl.when(2 * j + 3 < NCH)
                def _():
                    start(2 * j + 3, ra1, rb1, sa1, sb1)

            return carry

        lax.fori_loop(0, NPAIR, pair, 0)

    return k(hA, hB, sidx, ridx)


def _scatter_add(m, ridx):
    NPAIR = (NCH + 1) // 2

    @functools.partial(
        pl.kernel,
        out_type=[jax.ShapeDtypeStruct((NPAD, D), jnp.float32)] * 2,
        mesh=_SC_MESH,
        scratch_types=[
            pltpu.VMEM((C,), jnp.int32),
            pltpu.VMEM((C,), jnp.int32),
            pltpu.VMEM((C, D), jnp.float32),
            pltpu.VMEM((C, D), jnp.float32),
            pltpu.VMEM((ZR, D), jnp.float32),
            pltpu.VMEM_SHARED((NPAD, D), jnp.float32),
            pltpu.SemaphoreType.DMA,
            pltpu.SemaphoreType.DMA,
            pltpu.SemaphoreType.DMA,
            pltpu.SemaphoreType.DMA,
        ],
    )
    def k(m_h, r_h, o0, o1, iv0, iv1, mv0, mv1, zv, acc, si0, si1, sm0, sm1):
        c = lax.axis_index("c")
        s = lax.axis_index("s")
        wid = s * 2 + c
        base0 = wid * EW

        def start(i, iv, mv, si, sm):
            base = base0 + i * C
            pltpu.async_copy(r_h.at[pl.ds(base, C)], iv, si)
            pltpu.async_copy(m_h.at[pl.ds(base, C)], mv, sm)

        start(0, iv0, mv0, si0, sm0)
        start(1, iv1, mv1, si1, sm1)

        def zst(j, _):
            zv[j // 8, pl.ds((j % 8) * 16, 16)] = jnp.zeros((16,), jnp.float32)
            return _

        lax.fori_loop(0, ZR * 8, zst, 0)
        for t in range(SR // ZR):
            pltpu.sync_copy(zv, acc.at[pl.ds(s * SR + t * ZR, ZR)])
        plsc.subcore_barrier()

        def finish(i, iv, mv, si, sm):
            pltpu.make_async_copy(r_h.at[pl.ds(0, C)], iv, si).wait()
            pltpu.make_async_copy(m_h.at[pl.ds(0, C)], mv, sm).wait()
            pltpu.sync_copy(mv, acc.at[iv], add=True)

        def pair(j, carry):
            finish(2 * j, iv0, mv0, si0, sm0)

            @pl.when(2 * j + 2 < NCH)
            def _():
                start(2 * j + 2, iv0, mv0, si0, sm0)

            @pl.when(2 * j + 1 < NCH)
            def _():
                finish(2 * j + 1, iv1, mv1, si1, sm1)

                @pl.when(2 * j + 3 < NCH)
                def _():
                    start(2 * j + 3, iv1, mv1, si1, sm1)

            return carry

        lax.fori_loop(0, NPAIR, pair, 0)
        plsc.subcore_barrier()

        @pl.when(c == 0)
        def _():
            pltpu.sync_copy(acc.at[pl.ds(s * SR, SR)], o0.at[pl.ds(s * SR, SR)])

        @pl.when(c == 1)
        def _():
            pltpu.sync_copy(acc.at[pl.ds(s * SR, SR)], o1.at[pl.ds(s * SR, SR)])

    return k(m, ridx)


def kernel(x, edge_index, edge_attr, params):
    senders = edge_index[0].astype(jnp.int32)
    receivers = edge_index[1].astype(jnp.int32)

    l1, l2 = params["layers"]
    w1_1 = l1["edge_mlp"][0]["W"]
    w1_2 = l2["edge_mlp"][0]["W"]

    ne0, ne1 = params["node_enc"]
    h, hA, hB = _node_encoder(
        x, ne0["W"], ne0["b"], ne1["W"], ne1["b"],
        params["node_enc_ln"]["g"], params["node_enc_ln"]["b"],
        w1_1[:D], w1_1[D:2 * D], l1["edge_mlp"][0]["b"])
    ee0, ee1 = params["edge_enc"]
    ec1, ec2 = _edge_encoder(
        edge_attr, ee0["W"], ee0["b"], ee1["W"], ee1["b"],
        params["edge_enc_ln"]["g"], params["edge_enc_ln"]["b"],
        w1_1[2 * D:], w1_2[2 * D:])

    gsum = _gather_add(hA, hB, senders, receivers)
    m = _edge_mlp(gsum, ec1, l1["edge_mlp"][1]["W"], l1["edge_mlp"][1]["b"],
                  l1["edge_norm"]["g"], l1["edge_norm"]["b"])
    p0, p1 = _scatter_add(m, receivers)
    w3 = l1["node_mlp"][0]["W"]
    h, hA, hB = _node_update(
        h, p0, p1, w3[:D], w3[D:], l1["node_mlp"][0]["b"],
        l1["node_mlp"][1]["W"], l1["node_mlp"][1]["b"],
        l1["node_norm"]["g"], l1["node_norm"]["b"],
        w1_2[:D], w1_2[D:2 * D], l2["edge_mlp"][0]["b"])

    gsum = _gather_add(hA, hB, senders, receivers)
    m = _edge_mlp(gsum, ec2, l2["edge_mlp"][1]["W"], l2["edge_mlp"][1]["b"],
                  l2["edge_norm"]["g"], l2["edge_norm"]["b"])
    p0, p1 = _scatter_add(m, receivers)
    w3 = l2["node_mlp"][0]["W"]
    d0, d1 = params["dec"]
    return _node_update_dec(
        h, p0, p1, w3[:D], w3[D:], l2["node_mlp"][0]["b"],
        l2["node_mlp"][1]["W"], l2["node_mlp"][1]["b"],
        l2["node_norm"]["g"], l2["node_norm"]["b"],
        d0["W"], d0["b"], d1["W"], d1["b"])


# speedup vs baseline: 3.2763x; 1.1256x over previous
import functools

import jax
import jax.numpy as jnp
from jax import lax
from jax.experimental import pallas as pl
from jax.experimental.pallas import tpu as pltpu
from jax.experimental.pallas import tpu_sc as plsc

N = 10000
E = 320000
D = 128
NB = 1000
EB = 6400

NW = 32
EW = E // NW
C = 80
NCH = EW // C
NPAD = 10240
SR = NPAD // 16
ZR = 128

_PREC = jax.lax.Precision.HIGHEST


def _dot(a, b):
    return jnp.dot(a, b, preferred_element_type=jnp.float32, precision=_PREC)


def _ln(h, g, b):
    mu = jnp.mean(h, axis=-1, keepdims=True)
    d = h - mu
    var = jnp.mean(d * d, axis=-1, keepdims=True)
    return d * lax.rsqrt(var + 1e-5) * g + b


def _row(v):
    return v.reshape(1, -1)


def _full(shape):
    return pl.BlockSpec(shape, lambda i: (0,) * len(shape))


def _rows(blk, width):
    return pl.BlockSpec((blk, width), lambda i: (i, 0))


def _node_encoder(x, w1, b1, w2, b2, g, b, wab, bt):
    def body(x_r, w1_r, b1_r, w2_r, b2_r, g_r, b_r, wab_r, bt_r,
             o_r, oa_r, ob_r):
        h = jnp.maximum(_dot(x_r[...], w1_r[...]) + b1_r[...], 0.0)
        h = _dot(h, w2_r[...]) + b2_r[...]
        h = _ln(h, g_r[...], b_r[...])
        o_r[...] = h
        tab = _dot(h, wab_r[...])
        oa_r[...] = tab[:, :D]
        ob_r[...] = tab[:, D:] + bt_r[...]

    return pl.pallas_call(
        body,
        grid=(N // NB,),
        in_specs=[_rows(NB, D), _full((D, D)), _full((1, D)),
                  _full((D, D)), _full((1, D)), _full((1, D)), _full((1, D)),
                  _full((D, 2 * D)), _full((1, D))],
        out_specs=[_rows(NB, D)] * 3,
        out_shape=[jax.ShapeDtypeStruct((N, D), jnp.float32)] * 3,
    )(x, w1, _row(b1), w2, _row(b2), _row(g), _row(b), wab, _row(bt))


def _edge_encoder(ea_t, w1, b1, w2, b2, g, b, wc12):
    def body(ea_r, w1_r, b1_r, w2_r, b2_r, g_r, b_r, c_r, o1_r, o2_r):
        t = lax.dot_general(ea_r[...], w1_r[...], (((0,), (0,)), ((), ())),
                            preferred_element_type=jnp.float32,
                            precision=_PREC)
        t = jnp.maximum(t + b1_r[...], 0.0)
        e = _ln(_dot(t, w2_r[...]) + b2_r[...], g_r[...], b_r[...])
        ec = _dot(e, c_r[...])
        o1_r[...] = ec[:, :D]
        o2_r[...] = ec[:, D:]

    return pl.pallas_call(
        body,
        grid=(E // EB,),
        in_specs=[pl.BlockSpec((16, EB), lambda i: (0, i)), _full((16, D)),
                  _full((1, D)),
                  _full((D, D)), _full((1, D)), _full((1, D)), _full((1, D)),
                  _full((D, 2 * D))],
        out_specs=[_rows(EB, D)] * 2,
        out_shape=[jax.ShapeDtypeStruct((E, D), jnp.float32)] * 2,
    )(ea_t, w1, _row(b1), w2, _row(b2), _row(g), _row(b), wc12)


def _edge_mlp(gsum, ec, w2, b2, g, b):
    def body(g_r, e_r, w2_r, b2_r, gg_r, bb_r, o_r):
        x = jnp.maximum(g_r[...] + e_r[...], 0.0)
        m = jnp.maximum(_dot(x, w2_r[...]) + b2_r[...], 0.0)
        o_r[...] = _ln(m, gg_r[...], bb_r[...])

    return pl.pallas_call(
        body,
        grid=(E // EB,),
        in_specs=[_rows(EB, D), _rows(EB, D),
                  _full((D, D)), _full((1, D)), _full((1, D)), _full((1, D))],
        out_specs=_rows(EB, D),
        out_shape=jax.ShapeDtypeStruct((E, D), jnp.float32),
    )(gsum, ec, w2, _row(b2), _row(g), _row(b))


def _node_update(h, p0, p1, w3, b3, w4, b4, g, b, wab, bt):
    def body(h_r, p0_r, p1_r, w3_r, b3_r, w4_r, b4_r, gg_r, bb_r,
             tab_r, bt_r, o_r, oa_r, ob_r):
        hv = h_r[...]
        agg = p0_r[...] + p1_r[...]
        ni = jnp.concatenate([hv, agg], axis=1)
        nu = jnp.maximum(_dot(ni, w3_r[...]) + b3_r[...], 0.0)
        nu = _dot(nu, w4_r[...]) + b4_r[...]
        hn = _ln(hv + nu, gg_r[...], bb_r[...])
        o_r[...] = hn
        tab = _dot(hn, tab_r[...])
        oa_r[...] = tab[:, :D]
        ob_r[...] = tab[:, D:] + bt_r[...]

    return pl.pallas_call(
        body,
        grid=(N // NB,),
        in_specs=[_rows(NB, D), _rows(NB, D), _rows(NB, D), _full((2 * D, D)),
                  _full((1, D)), _full((D, D)), _full((1, D)),
                  _full((1, D)), _full((1, D)),
                  _full((D, 2 * D)), _full((1, D))],
        out_specs=[_rows(NB, D)] * 3,
        out_shape=[jax.ShapeDtypeStruct((N, D), jnp.float32)] * 3,
    )(h, p0, p1, w3, _row(b3), w4, _row(b4), _row(g), _row(b),
      wab, _row(bt))


def _node_update_dec(h, p0, p1, w3, b3, w4, b4, g, b, wd1, bd1, wd2, bd2):
    def body(h_r, p0_r, p1_r, w3_r, b3_r, w4_r, b4_r, gg_r, bb_r,
             d1_r, e1_r, d2_r, e2_r, o_r):
        hv = h_r[...]
        agg = p0_r[...] + p1_r[...]
        ni = jnp.concatenate([hv, agg], axis=1)
        nu = jnp.maximum(_dot(ni, w3_r[...]) + b3_r[...], 0.0)
        nu = _dot(nu, w4_r[...]) + b4_r[...]
        hn = _ln(hv + nu, gg_r[...], bb_r[...])
        z = jnp.maximum(_dot(hn, d1_r[...]) + e1_r[...], 0.0)
        o_r[...] = _dot(z, d2_r[...]) + e2_r[...]

    return pl.pallas_call(
        body,
        grid=(N // NB,),
        in_specs=[_rows(NB, D), _rows(NB, D), _rows(NB, D), _full((2 * D, D)),
                  _full((1, D)), _full((D, D)), _full((1, D)),
                  _full((1, D)), _full((1, D)),
                  _full((D, D)), _full((1, D)), _full((D, 3)), _full((1, 3))],
        out_specs=_rows(NB, 3),
        out_shape=jax.ShapeDtypeStruct((N, 3), jnp.float32),
    )(h, p0, p1, w3, _row(b3), w4, _row(b4), _row(g), _row(b),
      wd1, _row(bd1), wd2, _row(bd2))


_SC_MESH = plsc.VectorSubcoreMesh(core_axis_name="c", subcore_axis_name="s")


def _gather_add(hA, hB, sidx, ridx):
    NPAIR = (NCH + 1) // 2

    @functools.partial(
        pl.kernel,
        out_type=jax.ShapeDtypeStruct((E, D), jnp.float32),
        mesh=_SC_MESH,
        scratch_types=[
            pltpu.VMEM((EW,), jnp.int32),
            pltpu.VMEM((EW,), jnp.int32),
            pltpu.VMEM((C, D), jnp.float32),
            pltpu.VMEM((C, D), jnp.float32),
            pltpu.VMEM((C, D), jnp.float32),
            pltpu.VMEM((C, D), jnp.float32),
            pltpu.SemaphoreType.DMA,
            pltpu.SemaphoreType.DMA,
            pltpu.SemaphoreType.DMA,
            pltpu.SemaphoreType.DMA,
        ],
    )
    def k(hA_h, hB_h, s_h, r_h, out_h,
          sv, rv, ra0, rb0, ra1, rb1, sa0, sb0, sa1, sb1):
        wid = lax.axis_index("s") * 2 + lax.axis_index("c")
        base0 = wid * EW
        pltpu.sync_copy(s_h.at[pl.ds(base0, EW)], sv)
        pltpu.sync_copy(r_h.at[pl.ds(base0, EW)], rv)

        def start(i, ra, rb, sa, sb):
            off = i * C
            pltpu.async_copy(hA_h.at[sv.at[pl.ds(off, C)]], ra, sa)
            pltpu.async_copy(hB_h.at[rv.at[pl.ds(off, C)]], rb, sb)

        def finish(i, ra, rb, sa, sb):
            pltpu.make_async_copy(hA_h.at[pl.ds(0, C)], ra, sa).wait()
            pltpu.make_async_copy(hB_h.at[pl.ds(0, C)], rb, sb).wait()

            def rowadd(r, carry):
                for q in range(8):
                    sl = pl.ds(q * 16, 16)
                    ra[r, sl] = ra[r, sl] + rb[r, sl]
                return carry

            lax.fori_loop(0, C, rowadd, 0)
            pltpu.sync_copy(ra, out_h.at[pl.ds(base0 + i * C, C)])

        start(0, ra0, rb0, sa0, sb0)
        start(1, ra1, rb1, sa1, sb1)

        def pair(j, carry):
            finish(2 * j, ra0, rb0, sa0, sb0)

            @pl.when(2 * j + 2 < NCH)
            def _():
                start(2 * j + 2, ra0, rb0, sa0, sb0)

            @pl.when(2 * j + 1 < NCH)
            def _():
                finish(2 * j + 1, ra1, rb1, sa1, sb1)

                @pl.when(2 * j + 3 < NCH)
                def _():
                    start(2 * j + 3, ra1, rb1, sa1, sb1)

            return carry

        lax.fori_loop(0, NPAIR, pair, 0)

    return k(hA, hB, sidx, ridx)


def _scatter_add(m, ridx):
    NPAIR = (NCH + 1) // 2

    @functools.partial(
        pl.kernel,
        out_type=[jax.ShapeDtypeStruct((NPAD, D), jnp.float32)] * 2,
        mesh=_SC_MESH,
        scratch_types=[
            pltpu.VMEM((C,), jnp.int32),
            pltpu.VMEM((C,), jnp.int32),
            pltpu.VMEM((C, D), jnp.float32),
            pltpu.VMEM((C, D), jnp.float32),
            pltpu.VMEM((ZR, D), jnp.float32),
            pltpu.VMEM_SHARED((NPAD, D), jnp.float32),
            pltpu.SemaphoreType.DMA,
            pltpu.SemaphoreType.DMA,
            pltpu.SemaphoreType.DMA,
            pltpu.SemaphoreType.DMA,
        ],
    )
    def k(m_h, r_h, o0, o1, iv0, iv1, mv0, mv1, zv, acc, si0, si1, sm0, sm1):
        c = lax.axis_index("c")
        s = lax.axis_index("s")
        wid = s * 2 + c
        base0 = wid * EW

        def start(i, iv, mv, si, sm):
            base = base0 + i * C
            pltpu.async_copy(r_h.at[pl.ds(base, C)], iv, si)
            pltpu.async_copy(m_h.at[pl.ds(base, C)], mv, sm)

        start(0, iv0, mv0, si0, sm0)
        start(1, iv1, mv1, si1, sm1)

        def zst(j, _):
            zv[j // 8, pl.ds((j % 8) * 16, 16)] = jnp.zeros((16,), jnp.float32)
            return _

        lax.fori_loop(0, ZR * 8, zst, 0)
        for t in range(SR // ZR):
            pltpu.sync_copy(zv, acc.at[pl.ds(s * SR + t * ZR, ZR)])
        plsc.subcore_barrier()

        def finish(i, iv, mv, si, sm):
            pltpu.make_async_copy(r_h.at[pl.ds(0, C)], iv, si).wait()
            pltpu.make_async_copy(m_h.at[pl.ds(0, C)], mv, sm).wait()
            pltpu.sync_copy(mv, acc.at[iv], add=True)

        def pair(j, carry):
            finish(2 * j, iv0, mv0, si0, sm0)

            @pl.when(2 * j + 2 < NCH)
            def _():
                start(2 * j + 2, iv0, mv0, si0, sm0)

            @pl.when(2 * j + 1 < NCH)
            def _():
                finish(2 * j + 1, iv1, mv1, si1, sm1)

                @pl.when(2 * j + 3 < NCH)
                def _():
                    start(2 * j + 3, iv1, mv1, si1, sm1)

            return carry

        lax.fori_loop(0, NPAIR, pair, 0)
        plsc.subcore_barrier()

        @pl.when(c == 0)
        def _():
            pltpu.sync_copy(acc.at[pl.ds(s * SR, SR)], o0.at[pl.ds(s * SR, SR)])

        @pl.when(c == 1)
        def _():
            pltpu.sync_copy(acc.at[pl.ds(s * SR, SR)], o1.at[pl.ds(s * SR, SR)])

    return k(m, ridx)


def kernel(x, edge_index, edge_attr, params):
    senders = edge_index[0].astype(jnp.int32)
    receivers = edge_index[1].astype(jnp.int32)

    l1, l2 = params["layers"]
    w1_1 = l1["edge_mlp"][0]["W"]
    w1_2 = l2["edge_mlp"][0]["W"]

    ne0, ne1 = params["node_enc"]
    h, hA, hB = _node_encoder(
        x, ne0["W"], ne0["b"], ne1["W"], ne1["b"],
        params["node_enc_ln"]["g"], params["node_enc_ln"]["b"],
        jnp.concatenate([w1_1[:D], w1_1[D:2 * D]], axis=1),
        l1["edge_mlp"][0]["b"])
    ee0, ee1 = params["edge_enc"]
    ec1, ec2 = _edge_encoder(
        edge_attr.T, ee0["W"], ee0["b"], ee1["W"], ee1["b"],
        params["edge_enc_ln"]["g"], params["edge_enc_ln"]["b"],
        jnp.concatenate([w1_1[2 * D:], w1_2[2 * D:]], axis=1))

    gsum = _gather_add(hA, hB, senders, receivers)
    m = _edge_mlp(gsum, ec1, l1["edge_mlp"][1]["W"], l1["edge_mlp"][1]["b"],
                  l1["edge_norm"]["g"], l1["edge_norm"]["b"])
    p0, p1 = _scatter_add(m, receivers)
    h, hA, hB = _node_update(
        h, p0, p1, l1["node_mlp"][0]["W"], l1["node_mlp"][0]["b"],
        l1["node_mlp"][1]["W"], l1["node_mlp"][1]["b"],
        l1["node_norm"]["g"], l1["node_norm"]["b"],
        jnp.concatenate([w1_2[:D], w1_2[D:2 * D]], axis=1),
        l2["edge_mlp"][0]["b"])

    gsum = _gather_add(hA, hB, senders, receivers)
    m = _edge_mlp(gsum, ec2, l2["edge_mlp"][1]["W"], l2["edge_mlp"][1]["b"],
                  l2["edge_norm"]["g"], l2["edge_norm"]["b"])
    p0, p1 = _scatter_add(m, receivers)
    d0, d1 = params["dec"]
    return _node_update_dec(
        h, p0, p1, l2["node_mlp"][0]["W"], l2["node_mlp"][0]["b"],
        l2["node_mlp"][1]["W"], l2["node_mlp"][1]["b"],
        l2["node_norm"]["g"], l2["node_norm"]["b"],
        d0["W"], d0["b"], d1["W"], d1["b"])
